# Initial kernel scaffold; baseline (speedup 1.0000x reference)
import functools

import jax
import jax.numpy as jnp
from jax import lax
from jax.experimental import pallas as pl
from jax.experimental.pallas import tpu as pltpu
from jax.experimental.pallas import tpu_sc as plsc

B = 16384
H = 50
D = 64
O = 64
NC = 2
NS = 16
NW = NC * NS
RPW = B // NW
CB = 2
CHUNK = CB * H
NCHUNK = RPW // CB
NSLICE = D // 16


def _pooled_sc(x3, emb):
    mesh = plsc.VectorSubcoreMesh(core_axis_name="c", subcore_axis_name="s")

    @functools.partial(
        pl.kernel,
        mesh=mesh,
        out_type=jax.ShapeDtypeStruct((B, D), jnp.float32),
        scratch_types=[
            pltpu.VMEM((NCHUNK, CHUNK), jnp.int32),
            pltpu.VMEM((CHUNK, D), jnp.float32),
            pltpu.VMEM((RPW, D), jnp.float32),
            pltpu.SemaphoreType.DMA,
        ],
    )
    def k(x_hbm, emb_hbm, out_hbm, idx_v, buf_v, out_v, sem):
        wid = lax.axis_index("s") * NC + lax.axis_index("c")
        pltpu.sync_copy(x_hbm.at[wid], idx_v)

        def chunk_body(c, carry):
            pltpu.async_copy(emb_hbm.at[idx_v.at[c]], buf_v, sem).wait()
            for r in range(CB):
                def l_body(l, accs):
                    row = r * H + l
                    return tuple(
                        accs[j] + buf_v[row, pl.ds(j * 16, 16)]
                        for j in range(NSLICE)
                    )
                accs = lax.fori_loop(
                    0, H, l_body,
                    tuple(jnp.zeros((16,), jnp.float32) for _ in range(NSLICE)),
                )
                for j in range(NSLICE):
                    out_v[c * CB + r, pl.ds(j * 16, 16)] = accs[j] * (1.0 / H)
            return carry

        lax.fori_loop(0, NCHUNK, chunk_body, 0)
        pltpu.sync_copy(out_v, out_hbm.at[pl.ds(wid * RPW, RPW)])

    return k(x3, emb)


def _linear_tc(pooled, W, b):
    BM = 2048

    def mm(p_ref, w_ref, b_ref, o_ref):
        o_ref[...] = (
            jnp.dot(p_ref[...], w_ref[...], preferred_element_type=jnp.float32)
            + b_ref[...]
        )

    return pl.pallas_call(
        mm,
        grid=(B // BM,),
        in_specs=[
            pl.BlockSpec((BM, D), lambda i: (i, 0)),
            pl.BlockSpec((D, O), lambda i: (0, 0)),
            pl.BlockSpec((1, O), lambda i: (0, 0)),
        ],
        out_specs=pl.BlockSpec((BM, O), lambda i: (i, 0)),
        out_shape=jax.ShapeDtypeStruct((B, O), jnp.float32),
    )(pooled, W, b.reshape(1, O))


def kernel(x, emb, W, b):
    x3 = x.astype(jnp.int32).reshape(NW, NCHUNK, CHUNK)
    pooled = _pooled_sc(x3, emb)
    return _linear_tc(pooled, W, b)


# speedup vs baseline: 2.5127x; 2.5127x over previous
import functools

import jax
import jax.numpy as jnp
from jax import lax
from jax.experimental import pallas as pl
from jax.experimental.pallas import tpu as pltpu
from jax.experimental.pallas import tpu_sc as plsc

B = 16384
H = 50
D = 64
O = 64
NC = 2
NS = 16
NW = NC * NS
RPW = B // NW
CB = 2
CHUNK = CB * H
NCHUNK = RPW // CB
NSLICE = D // 16


def _pooled_sc(x3, emb):
    mesh = plsc.VectorSubcoreMesh(core_axis_name="c", subcore_axis_name="s")

    @functools.partial(
        pl.kernel,
        mesh=mesh,
        out_type=jax.ShapeDtypeStruct((B, D), jnp.float32),
        compiler_params=pltpu.CompilerParams(use_tc_tiling_on_sc=False),
        scratch_types=[
            pltpu.VMEM((NCHUNK, CHUNK), jnp.int32),
            pltpu.VMEM((2, CHUNK, D), jnp.float32),
            pltpu.VMEM((RPW, D), jnp.float32),
            pltpu.SemaphoreType.DMA,
            pltpu.SemaphoreType.DMA,
        ],
    )
    def k(x_hbm, emb_hbm, out_hbm, idx_v, buf_v, out_v, sem0, sem1):
        wid = lax.axis_index("s") * NC + lax.axis_index("c")
        sems = (sem0, sem1)
        pltpu.sync_copy(x_hbm.at[wid], idx_v)

        def start(chunk, s):
            pltpu.async_copy(emb_hbm.at[idx_v.at[chunk]], buf_v.at[s], sems[s])

        def wait(s):
            pltpu.make_async_copy(
                emb_hbm.at[idx_v.at[0]], buf_v.at[s], sems[s]
            ).wait()

        start(0, 0)
        start(1, 1)

        def pair_body(i, carry):
            for s in range(2):
                chunk = 2 * i + s
                wait(s)
                for r in range(CB):
                    accs = [buf_v[s, r * H, pl.ds(j * 16, 16)]
                            for j in range(NSLICE)]
                    for l in range(1, H):
                        for j in range(NSLICE):
                            accs[j] = accs[j] + buf_v[s, r * H + l,
                                                      pl.ds(j * 16, 16)]
                    for j in range(NSLICE):
                        out_v[chunk * CB + r, pl.ds(j * 16, 16)] = (
                            accs[j] * (1.0 / H)
                        )

                @pl.when(chunk + 2 < NCHUNK)
                def _():
                    start(chunk + 2, s)
            return carry

        lax.fori_loop(0, NCHUNK // 2, pair_body, 0)
        pltpu.sync_copy(out_v, out_hbm.at[pl.ds(wid * RPW, RPW)])

    return k(x3, emb)


def _linear_tc(pooled, W, b):
    BM = 2048

    def mm(p_ref, w_ref, b_ref, o_ref):
        o_ref[...] = (
            jnp.dot(p_ref[...], w_ref[...], preferred_element_type=jnp.float32)
            + b_ref[...]
        )

    return pl.pallas_call(
        mm,
        grid=(B // BM,),
        in_specs=[
            pl.BlockSpec((BM, D), lambda i: (i, 0)),
            pl.BlockSpec((D, O), lambda i: (0, 0)),
            pl.BlockSpec((1, O), lambda i: (0, 0)),
        ],
        out_specs=pl.BlockSpec((BM, O), lambda i: (i, 0)),
        out_shape=jax.ShapeDtypeStruct((B, O), jnp.float32),
    )(pooled, W, b.reshape(1, O))


def kernel(x, emb, W, b):
    x3 = x.astype(jnp.int32).reshape(NW, NCHUNK, CHUNK)
    pooled = _pooled_sc(x3, emb)
    return _linear_tc(pooled, W, b)
